# baseline (speedup 1.0000x reference)
import functools

import jax
import jax.lax as lax
import jax.numpy as jnp
from jax.experimental import pallas as pl
from jax.experimental.pallas import tpu as pltpu
from jax.experimental.pallas import tpu_sc as plsc

N = 10000
E = 160000
L = 320000
B = 256
D = 128
ED = 6
NITER = 4

CH = 8192
NCHUNK = 20
EPAD = CH * NCHUNK
CPC = NCHUNK // 2
KBLK = 128
NSUB = 16
RPS = CH // NSUB
BPS = RPS // KBLK

@functools.cache
def _sc_mesh():
    return plsc.VectorSubcoreMesh(core_axis_name="c", subcore_axis_name="s",
                                  num_cores=2, num_subcores=NSUB)


def _pre_node_body(x_ref, wmlp_ref, bmlp_ref, wu_ref, wv_ref,
                   h_ref, eu_ref, ev_ref):
    h = jnp.dot(x_ref[...], wmlp_ref[...],
                preferred_element_type=jnp.float32) + bmlp_ref[...]
    h_ref[...] = h
    third = jnp.float32(1.0 / 3.0)
    eu_ref[...] = jnp.dot(h, wu_ref[...],
                          preferred_element_type=jnp.float32) * third
    ev_ref[...] = jnp.dot(h, wv_ref[...],
                          preferred_element_type=jnp.float32) * third


def _pre_node(x, W_mlp, b_mlp, W_u, W_v):
    blk = 1000
    return pl.pallas_call(
        _pre_node_body,
        grid=(N // blk,),
        in_specs=[
            pl.BlockSpec((blk, D), lambda i: (i, 0)),
            pl.BlockSpec((D, D), lambda i: (0, 0)),
            pl.BlockSpec((1, D), lambda i: (0, 0)),
            pl.BlockSpec((D, D), lambda i: (0, 0)),
            pl.BlockSpec((D, D), lambda i: (0, 0)),
        ],
        out_specs=[
            pl.BlockSpec((blk, D), lambda i: (i, 0)),
            pl.BlockSpec((blk, D), lambda i: (i, 0)),
            pl.BlockSpec((blk, D), lambda i: (i, 0)),
        ],
        out_shape=[jax.ShapeDtypeStruct((N, D), jnp.float32)] * 3,
    )(x, W_mlp, b_mlp.reshape(1, D), W_u, W_v)


def _pre_edge_body(ea_ref, we_ref, euv_ref):
    euv_ref[...] = jnp.dot(ea_ref[...], we_ref[...],
                           preferred_element_type=jnp.float32) * jnp.float32(1.0 / 3.0)


def _pre_edge(edge_attr, W_edge):
    blk = 2048
    ea = jnp.pad(edge_attr, ((0, EPAD - E), (0, 8 - ED)))
    we = jnp.pad(W_edge, ((0, 8 - ED), (0, 0)))
    return pl.pallas_call(
        _pre_edge_body,
        grid=(EPAD // blk,),
        in_specs=[
            pl.BlockSpec((blk, 8), lambda i: (i, 0)),
            pl.BlockSpec((8, D), lambda i: (0, 0)),
        ],
        out_specs=pl.BlockSpec((blk, D), lambda i: (i, 0)),
        out_shape=jax.ShapeDtypeStruct((EPAD, D), jnp.float32),
    )(ea, we)


def _final_body(xo_ref, w_ref, b_ref, o_ref):
    o_ref[...] = jnp.dot(xo_ref[...], w_ref[...],
                         preferred_element_type=jnp.float32) + b_ref[...]


def _final_linear(xo_pad, W_lb, b_lb):
    blk = 1000
    return pl.pallas_call(
        _final_body,
        grid=(N // blk,),
        in_specs=[
            pl.BlockSpec((blk, D), lambda i: (i, 0)),
            pl.BlockSpec((D, D), lambda i: (0, 0)),
            pl.BlockSpec((1, D), lambda i: (0, 0)),
        ],
        out_specs=pl.BlockSpec((blk, D), lambda i: (i, 0)),
        out_shape=jax.ShapeDtypeStruct((N, D), jnp.float32),
    )(xo_pad, W_lb, b_lb.reshape(1, D))


def _e0_body(table, gidx_blocks, iota_rows, init, out,
             acc, idx_v, ldst_v, rows_v, sem):
    core = lax.axis_index("c")
    s = lax.axis_index("s")
    pltpu.sync_copy(iota_rows.at[pl.ds(s * BPS, BPS)], ldst_v)

    def chunk_body(cb, carry):
        b = core * CPC + cb
        row0 = b * CH + s * RPS
        pltpu.sync_copy(init.at[pl.ds(row0, RPS)], acc.at[pl.ds(s * RPS, RPS)])
        plsc.subcore_barrier()

        def part_body(p, carry2):
            def blk_body(jj, carry3):
                blkrow = p * (EPAD // KBLK) + b * (CH // KBLK) + s * BPS + jj
                pltpu.sync_copy(gidx_blocks.at[blkrow], idx_v)
                pltpu.async_copy(table.at[idx_v], rows_v, sem).wait()
                pltpu.sync_copy(rows_v, acc.at[ldst_v.at[jj]], add=True)
                return carry3
            return lax.fori_loop(0, BPS, blk_body, carry2)

        lax.fori_loop(0, 2, part_body, 0)
        plsc.subcore_barrier()
        pltpu.sync_copy(acc.at[pl.ds(s * RPS, RPS)], out.at[pl.ds(row0, RPS)])
        plsc.subcore_barrier()
        return carry

    lax.fori_loop(0, CPC, chunk_body, 0)


@functools.partial(jax.jit, static_argnames=())
def _sc_e0(table, gidx_blocks, iota_rows, init):
    return pl.kernel(
        _e0_body,
        out_type=jax.ShapeDtypeStruct((EPAD, D), jnp.float32),
        mesh=_sc_mesh(),
        scratch_types=[
            pltpu.VMEM_SHARED((CH, D), jnp.float32),
            pltpu.VMEM((KBLK,), jnp.int32),
            pltpu.VMEM((BPS, KBLK), jnp.int32),
            pltpu.VMEM((KBLK, D), jnp.float32),
            pltpu.SemaphoreType.DMA,
        ],
    )(table, gidx_blocks, iota_rows, init)


def _make_count(total, nbuckets):
    per_w = total // 32
    cblk = 2000
    nload = per_w // cblk

    def body(keys, out, cnt_v, buf):
        core = lax.axis_index("c")
        s = lax.axis_index("s")
        wid = s * 2 + core
        zeros16 = jnp.zeros((16,), jnp.int32)
        for b in range(nbuckets):
            cnt_v[pl.ds(b * 16, 16)] = zeros16

        def load_body(c5, carry):
            pltpu.sync_copy(keys.at[pl.ds(wid * per_w + c5 * cblk, cblk)], buf)

            def vec_body(k, carry2):
                dv = buf[pl.ds(k * 16, 16)]
                cid = lax.shift_right_logical(dv, 13)
                for b in range(nbuckets):
                    cnt_v[pl.ds(b * 16, 16)] = (
                        cnt_v[pl.ds(b * 16, 16)]
                        + jnp.where(cid == b, 1, 0))
                return carry2

            return lax.fori_loop(0, cblk // 16, vec_body, carry)

        lax.fori_loop(0, nload, load_body, 0)
        pltpu.sync_copy(cnt_v, out.at[wid])

    def run(keys):
        lanes = pl.kernel(
            body,
            out_type=jax.ShapeDtypeStruct((32, nbuckets * 16), jnp.int32),
            mesh=_sc_mesh(),
            scratch_types=[
                pltpu.VMEM((nbuckets * 16,), jnp.int32),
                pltpu.VMEM((cblk,), jnp.int32),
            ],
        )(keys)
        counts = lanes.reshape(32, nbuckets, 16).sum(-1)
        return jnp.zeros((32, 32), jnp.int32).at[:, :nbuckets].set(counts)

    return run


_count_lg = _make_count(L, NCHUNK)
_count_e = _make_count(E, 2)


def _make_fill(total, nbuckets, payload_iota, dummy1, dummy2, shift2_thresh):
    per_w = total // 32
    cblk = 2000
    nload = per_w // cblk
    ncapb = total // KBLK + 32 * nbuckets

    def body(keys, payload, starts_w, o_src, o_d1, o_d2,
             rowbuf, posblk_v, fill_v,
             stg_s, stg_1, stg_2, kbuf, pbuf):
        core = lax.axis_index("c")
        s = lax.axis_index("s")
        wid = s * 2 + core
        pltpu.sync_copy(starts_w.at[wid], rowbuf)
        st_lo = rowbuf[pl.ds(0, 16)]
        st_hi = rowbuf[pl.ds(16, 16)]
        for b in range(nbuckets):
            stv = st_lo[b] if b < 16 else st_hi[b - 16]
            posblk_v[b] = stv
            fill_v[b] = 0

        dummy_s = jnp.zeros((16,), jnp.int32)
        dummy_1 = jnp.full((16,), dummy1, jnp.int32)
        dummy_2 = jnp.full((16,), dummy2, jnp.int32)

        SW = KBLK + 16

        def flush(b, pos):
            pltpu.sync_copy(stg_s.at[pl.ds(b * SW, KBLK)],
                            o_src.at[pl.ds(pos * KBLK, KBLK)])
            pltpu.sync_copy(stg_1.at[pl.ds(b * SW, KBLK)],
                            o_d1.at[pl.ds(pos * KBLK, KBLK)])
            pltpu.sync_copy(stg_2.at[pl.ds(b * SW, KBLK)],
                            o_d2.at[pl.ds(pos * KBLK, KBLK)])

        def load_body(c5, carry):
            off = wid * per_w + c5 * cblk
            pltpu.sync_copy(keys.at[pl.ds(off, cblk)], kbuf)
            if not payload_iota:
                pltpu.sync_copy(payload.at[pl.ds(off, cblk)], pbuf)

            def vec_body(k, carry2):
                dv = kbuf[pl.ds(k * 16, 16)]
                if payload_iota:
                    pv = lax.iota(jnp.int32, 16) + (off + k * 16)
                else:
                    pv = pbuf[pl.ds(k * 16, 16)]
                cid = lax.shift_right_logical(dv, 13)
                l1 = dv - lax.shift_left(cid, 13)
                l2 = jnp.where(cid >= shift2_thresh,
                               dv - shift2_thresh * CH, dv)
                lane = lax.iota(jnp.int32, 16)
                for b in range(nbuckets):
                    msk = cid == b
                    cntv = jnp.where(msk, 1, 0)
                    for sh in (8, 4, 2, 1):
                        cntv = cntv + cntv.at[lane ^ sh].get(
                            mode="promise_in_bounds")
                    cnt = cntv[0]

                    @pl.when(cnt > 0)
                    def _():
                        fill = fill_v[b]
                        o = b * SW + fill
                        plsc.store_compressed(stg_s.at[pl.ds(o, 16)], pv, mask=msk)
                        plsc.store_compressed(stg_1.at[pl.ds(o, 16)], l1, mask=msk)
                        plsc.store_compressed(stg_2.at[pl.ds(o, 16)], l2, mask=msk)
                        nf = fill + cnt

                        @pl.when(nf >= KBLK)
                        def _():
                            flush(b, posblk_v[b])
                            posblk_v[b] = posblk_v[b] + 1
                            tail_s = stg_s[pl.ds(b * SW + KBLK, 16)]
                            tail_1 = stg_1[pl.ds(b * SW + KBLK, 16)]
                            tail_2 = stg_2[pl.ds(b * SW + KBLK, 16)]
                            stg_s[pl.ds(b * SW, 16)] = tail_s
                            stg_1[pl.ds(b * SW, 16)] = tail_1
                            stg_2[pl.ds(b * SW, 16)] = tail_2

                        fill_v[b] = lax.rem(nf, jnp.int32(KBLK))
                return carry2

            return lax.fori_loop(0, cblk // 16, vec_body, carry)

        lax.fori_loop(0, nload, load_body, 0)

        for b in range(nbuckets):
            fill = fill_v[b]

            @pl.when(fill > 0)
            def _():
                def pad_body(j, carry3):
                    pos = fill + j * 16

                    @pl.when(pos < KBLK)
                    def _():
                        stg_s[pl.ds(b * SW + pos, 16)] = dummy_s
                        stg_1[pl.ds(b * SW + pos, 16)] = dummy_1
                        stg_2[pl.ds(b * SW + pos, 16)] = dummy_2

                    return carry3

                lax.fori_loop(0, 8, pad_body, 0)
                flush(b, posblk_v[b])

    def run(keys, payload, starts_w):
        return pl.kernel(
            body,
            out_type=[jax.ShapeDtypeStruct((ncapb * KBLK,), jnp.int32)] * 3,
            mesh=_sc_mesh(),
            compiler_params=pltpu.CompilerParams(needs_layout_passes=False),
            scratch_types=[
                pltpu.VMEM((32,), jnp.int32),
                pltpu.SMEM((32,), jnp.int32),
                pltpu.SMEM((32,), jnp.int32),
                pltpu.VMEM((nbuckets * (KBLK + 16),), jnp.int32),
                pltpu.VMEM((nbuckets * (KBLK + 16),), jnp.int32),
                pltpu.VMEM((nbuckets * (KBLK + 16),), jnp.int32),
                pltpu.VMEM((cblk,), jnp.int32),
                pltpu.VMEM((cblk,), jnp.int32),
            ],
        )(keys, payload, starts_w)

    return run


_fill_lg = _make_fill(L, NCHUNK, False, CH, CPC * CH, CPC)
_fill_e = _make_fill(E, 2, True, CH, CH, 1)


def _make_msg(cpc, out_rows):
    def body(table, init, meta_nb, meta_st, bsrc_b, bldst_b, out,
             acc, mrow, idx_v, ldst_v, rows_v, sem):
        core = lax.axis_index("c")
        s = lax.axis_index("s")
        wid = s * 2 + core
        pltpu.sync_copy(meta_nb.at[wid], mrow)
        nb_lo = mrow[pl.ds(0, 16)]
        nb_hi = mrow[pl.ds(16, 16)]
        pltpu.sync_copy(meta_st.at[wid], mrow)
        st_lo = mrow[pl.ds(0, 16)]
        st_hi = mrow[pl.ds(16, 16)]

        for cb in range(cpc):
            b = core * cpc + cb
            row0 = b * CH + s * RPS
            pltpu.sync_copy(init.at[pl.ds(row0, RPS)],
                            acc.at[pl.ds(s * RPS, RPS)])
            plsc.subcore_barrier()
            for tt in range(2):
                k = cb * 2 + tt
                nblk = nb_lo[k] if k < 16 else nb_hi[k - 16]
                st = st_lo[k] if k < 16 else st_hi[k - 16]

                def blk_body(j, carry2, st=st):
                    posrow = st + j
                    pltpu.sync_copy(bsrc_b.at[pl.ds(posrow * KBLK, KBLK)], idx_v)
                    pltpu.sync_copy(bldst_b.at[pl.ds(posrow * KBLK, KBLK)], ldst_v)
                    pltpu.async_copy(table.at[idx_v], rows_v, sem).wait()
                    pltpu.sync_copy(rows_v, acc.at[ldst_v], add=True)
                    return carry2

                lax.fori_loop(0, nblk, blk_body, 0)
            plsc.subcore_barrier()
            pltpu.sync_copy(acc.at[pl.ds(s * RPS, RPS)],
                            out.at[pl.ds(row0, RPS)])
            plsc.subcore_barrier()

    def run(table, init, meta_nb, meta_st, bsrc_b, bldst_b):
        return pl.kernel(
            body,
            out_type=jax.ShapeDtypeStruct((out_rows, D), jnp.float32),
            mesh=_sc_mesh(),
            scratch_types=[
                pltpu.VMEM_SHARED((CH + 8, D), jnp.float32),
                pltpu.VMEM((32,), jnp.int32),
                pltpu.VMEM((KBLK,), jnp.int32),
                pltpu.VMEM((KBLK,), jnp.int32),
                pltpu.VMEM((KBLK, D), jnp.float32),
                pltpu.SemaphoreType.DMA,
            ],
        )(table, init, meta_nb, meta_st, bsrc_b, bldst_b)

    return run


_msg_lg = _make_msg(CPC, EPAD)
_msg_fin = _make_msg(1, 2 * CH)


def _route_meta(counts, nbuckets, cpc):
    caps = (counts + 127) // KBLK
    capsT = caps.T[:nbuckets]
    flat = capsT.reshape(-1)
    starts_flat = jnp.cumsum(flat) - flat
    starts_bw = starts_flat.reshape(nbuckets, 32)
    starts_w = jnp.zeros((32, 32), jnp.int32).at[:, :nbuckets].set(
        starts_bw.T.astype(jnp.int32))
    w = jnp.arange(32)
    kk = jnp.arange(2 * cpc)
    cb = kk // 2
    tt = kk % 2
    t_idx = 2 * (w[:, None] // 2) + tt[None, :]
    b_idx = (w[:, None] % 2) * cpc + cb[None, :]
    meta_nb = jnp.zeros((32, 32), jnp.int32).at[:, :2 * cpc].set(
        caps[t_idx, b_idx].astype(jnp.int32))
    meta_st = jnp.zeros((32, 32), jnp.int32).at[:, :2 * cpc].set(
        starts_bw[b_idx, t_idx].astype(jnp.int32))
    return starts_w, meta_nb, meta_st


def kernel(x, edge_attr, edge_index, line_graph_edge_index, edge_index_batch,
           W_mlp, b_mlp, W_u, W_v, W_edge, W_att_root, W_att_rel, b_att, a,
           W_gout, b_gout, a_bias, W_lb, b_lb):
    h, eu3, ev3 = _pre_node(x, W_mlp, b_mlp, W_u, W_v)
    euv3 = _pre_edge(edge_attr, W_edge)

    src = edge_index[0].astype(jnp.int32)
    dst = edge_index[1].astype(jnp.int32)
    lg_src = line_graph_edge_index[0].astype(jnp.int32)
    lg_dst = line_graph_edge_index[1].astype(jnp.int32)
    batch = edge_index_batch.astype(jnp.int32)

    table = jnp.concatenate([eu3, ev3], axis=0)
    src_pad = jnp.pad(src, (0, EPAD - E))
    dst_pad = jnp.pad(dst, (0, EPAD - E))
    gidx_blocks = jnp.concatenate([src_pad, dst_pad + N]).reshape(-1, KBLK)
    iota_rows = jnp.arange(CH, dtype=jnp.int32).reshape(CH // KBLK, KBLK)

    e0_pad = _sc_e0(table, gidx_blocks, iota_rows, euv3)
    e0 = e0_pad[:E]

    counts_lg = _count_lg(lg_dst)
    starts_w, meta_nb, meta_st = _route_meta(counts_lg, NCHUNK, CPC)
    bsrc_b, bldst_b, bldst2_b = _fill_lg(lg_dst, lg_src, starts_w)

    out_pad = e0_pad
    out_list = []
    gout_list = []
    for n in range(NITER):
        out_pad = _msg_lg(out_pad, e0_pad, meta_nb, meta_st, bsrc_b, bldst_b)
        out = out_pad[:E]
        s = out @ W_att_rel
        nbs = jax.ops.segment_sum(s[lg_src], lg_dst, num_segments=E)
        xc = out @ W_att_root + nbs + b_att
        m = jax.ops.segment_max(xc, batch, num_segments=B)
        ex = jnp.exp(xc - m[batch])
        den = jax.ops.segment_sum(ex, batch, num_segments=B)
        sc = ex / den[batch]
        gx = jax.ops.segment_sum(out * sc, batch, num_segments=B)
        out_list.append(out)
        gout_list.append(jnp.tanh(gx @ W_gout + b_gout))
    gout_all = jnp.stack(gout_list, axis=-1)
    out_all = jnp.stack(out_list, axis=-1)
    scores = jnp.sum(gout_all * a, axis=1, keepdims=True) + a_bias
    scores = jax.nn.softmax(scores, axis=-1)
    scores_e = scores[batch]
    out_fin = jnp.sum(out_all * scores_e, axis=-1)
    xo = h + jax.ops.segment_sum(out_fin, dst, num_segments=N)
    return _final_linear(xo, W_lb, b_lb)


# speedup vs baseline: 1.9237x; 1.9237x over previous
import functools

import jax
import jax.lax as lax
import jax.numpy as jnp
from jax.experimental import pallas as pl
from jax.experimental.pallas import tpu as pltpu
from jax.experimental.pallas import tpu_sc as plsc

N = 10000
E = 160000
L = 320000
B = 256
D = 128
ED = 6
NITER = 4

CH = 8192
NCHUNK = 20
EPAD = CH * NCHUNK
CPC = NCHUNK // 2
KBLK = 128
NSUB = 16
RPS = CH // NSUB
BPS = RPS // KBLK

@functools.cache
def _sc_mesh():
    return plsc.VectorSubcoreMesh(core_axis_name="c", subcore_axis_name="s",
                                  num_cores=2, num_subcores=NSUB)


def _pre_node_body(x_ref, wmlp_ref, bmlp_ref, wu_ref, wv_ref,
                   h_ref, eu_ref, ev_ref):
    h = jnp.dot(x_ref[...], wmlp_ref[...],
                preferred_element_type=jnp.float32) + bmlp_ref[...]
    h_ref[...] = h
    third = jnp.float32(1.0 / 3.0)
    eu_ref[...] = jnp.dot(h, wu_ref[...],
                          preferred_element_type=jnp.float32) * third
    ev_ref[...] = jnp.dot(h, wv_ref[...],
                          preferred_element_type=jnp.float32) * third


def _pre_node(x, W_mlp, b_mlp, W_u, W_v):
    blk = 1000
    return pl.pallas_call(
        _pre_node_body,
        grid=(N // blk,),
        in_specs=[
            pl.BlockSpec((blk, D), lambda i: (i, 0)),
            pl.BlockSpec((D, D), lambda i: (0, 0)),
            pl.BlockSpec((1, D), lambda i: (0, 0)),
            pl.BlockSpec((D, D), lambda i: (0, 0)),
            pl.BlockSpec((D, D), lambda i: (0, 0)),
        ],
        out_specs=[
            pl.BlockSpec((blk, D), lambda i: (i, 0)),
            pl.BlockSpec((blk, D), lambda i: (i, 0)),
            pl.BlockSpec((blk, D), lambda i: (i, 0)),
        ],
        out_shape=[jax.ShapeDtypeStruct((N, D), jnp.float32)] * 3,
    )(x, W_mlp, b_mlp.reshape(1, D), W_u, W_v)


def _pre_edge_body(ea_ref, we_ref, euv_ref):
    euv_ref[...] = jnp.dot(ea_ref[...], we_ref[...],
                           preferred_element_type=jnp.float32) * jnp.float32(1.0 / 3.0)


def _pre_edge(edge_attr, W_edge):
    blk = 2048
    ea = jnp.pad(edge_attr, ((0, EPAD - E), (0, 8 - ED)))
    we = jnp.pad(W_edge, ((0, 8 - ED), (0, 0)))
    return pl.pallas_call(
        _pre_edge_body,
        grid=(EPAD // blk,),
        in_specs=[
            pl.BlockSpec((blk, 8), lambda i: (i, 0)),
            pl.BlockSpec((8, D), lambda i: (0, 0)),
        ],
        out_specs=pl.BlockSpec((blk, D), lambda i: (i, 0)),
        out_shape=jax.ShapeDtypeStruct((EPAD, D), jnp.float32),
    )(ea, we)


def _final_body(xo_ref, w_ref, b_ref, o_ref):
    o_ref[...] = jnp.dot(xo_ref[...], w_ref[...],
                         preferred_element_type=jnp.float32) + b_ref[...]


def _final_linear(xo_pad, W_lb, b_lb):
    blk = 1000
    return pl.pallas_call(
        _final_body,
        grid=(N // blk,),
        in_specs=[
            pl.BlockSpec((blk, D), lambda i: (i, 0)),
            pl.BlockSpec((D, D), lambda i: (0, 0)),
            pl.BlockSpec((1, D), lambda i: (0, 0)),
        ],
        out_specs=pl.BlockSpec((blk, D), lambda i: (i, 0)),
        out_shape=jax.ShapeDtypeStruct((N, D), jnp.float32),
    )(xo_pad, W_lb, b_lb.reshape(1, D))


def _e0_body(table, gidx_blocks, iota_rows, init, out,
             acc, idx_v, ldst_v, rows_v, sem):
    core = lax.axis_index("c")
    s = lax.axis_index("s")
    pltpu.sync_copy(iota_rows.at[pl.ds(s * BPS, BPS)], ldst_v)

    def chunk_body(cb, carry):
        b = core * CPC + cb
        row0 = b * CH + s * RPS
        pltpu.sync_copy(init.at[pl.ds(row0, RPS)], acc.at[pl.ds(s * RPS, RPS)])
        plsc.subcore_barrier()

        def part_body(p, carry2):
            def blk_body(jj, carry3):
                blkrow = p * (EPAD // KBLK) + b * (CH // KBLK) + s * BPS + jj
                pltpu.sync_copy(gidx_blocks.at[blkrow], idx_v)
                pltpu.async_copy(table.at[idx_v], rows_v, sem).wait()
                pltpu.sync_copy(rows_v, acc.at[ldst_v.at[jj]], add=True)
                return carry3
            return lax.fori_loop(0, BPS, blk_body, carry2)

        lax.fori_loop(0, 2, part_body, 0)
        plsc.subcore_barrier()
        pltpu.sync_copy(acc.at[pl.ds(s * RPS, RPS)], out.at[pl.ds(row0, RPS)])
        plsc.subcore_barrier()
        return carry

    lax.fori_loop(0, CPC, chunk_body, 0)


@functools.partial(jax.jit, static_argnames=())
def _sc_e0(table, gidx_blocks, iota_rows, init):
    return pl.kernel(
        _e0_body,
        out_type=jax.ShapeDtypeStruct((EPAD, D), jnp.float32),
        mesh=_sc_mesh(),
        scratch_types=[
            pltpu.VMEM_SHARED((CH, D), jnp.float32),
            pltpu.VMEM((KBLK,), jnp.int32),
            pltpu.VMEM((BPS, KBLK), jnp.int32),
            pltpu.VMEM((KBLK, D), jnp.float32),
            pltpu.SemaphoreType.DMA,
        ],
    )(table, gidx_blocks, iota_rows, init)


def _make_count(total, nbuckets, cblk):
    per_w = total // 32
    nload = per_w // cblk

    def body(keys, out, cnt_v, buf):
        core = lax.axis_index("c")
        s = lax.axis_index("s")
        wid = s * 2 + core
        zeros16 = jnp.zeros((16,), jnp.int32)
        for b in range(nbuckets):
            cnt_v[pl.ds(b * 16, 16)] = zeros16

        def load_body(c5, carry):
            pltpu.sync_copy(keys.at[pl.ds(wid * per_w + c5 * cblk, cblk)], buf)

            def vec_body(k, carry2):
                dv = buf[pl.ds(k * 16, 16)]
                cid = lax.shift_right_logical(dv, 13)
                for b in range(nbuckets):
                    cnt_v[pl.ds(b * 16, 16)] = (
                        cnt_v[pl.ds(b * 16, 16)]
                        + jnp.where(cid == b, 1, 0))
                return carry2

            return lax.fori_loop(0, cblk // 16, vec_body, carry)

        lax.fori_loop(0, nload, load_body, 0)
        pltpu.sync_copy(cnt_v, out.at[wid])

    def run(keys):
        lanes = pl.kernel(
            body,
            out_type=jax.ShapeDtypeStruct((32, nbuckets * 16), jnp.int32),
            mesh=_sc_mesh(),
            scratch_types=[
                pltpu.VMEM((nbuckets * 16,), jnp.int32),
                pltpu.VMEM((cblk,), jnp.int32),
            ],
        )(keys)
        counts = lanes.reshape(32, nbuckets, 16).sum(-1)
        return jnp.zeros((32, 32), jnp.int32).at[:, :nbuckets].set(counts)

    return run


_count_lg = _make_count(L, NCHUNK, 2000)
_count_e = _make_count(EPAD, 2, 2560)


def _make_fill(total, nbuckets, payload_iota, dummy1, dummy2, shift2_thresh, cblk):
    per_w = total // 32
    nload = per_w // cblk
    ncapb = total // KBLK + 32 * nbuckets

    def body(keys, payload, starts_w, o_src, o_d1, o_d2,
             rowbuf, posblk_v, fill_v,
             stg_s, stg_1, stg_2, kbuf, pbuf):
        core = lax.axis_index("c")
        s = lax.axis_index("s")
        wid = s * 2 + core
        pltpu.sync_copy(starts_w.at[wid], rowbuf)
        st_lo = rowbuf[pl.ds(0, 16)]
        st_hi = rowbuf[pl.ds(16, 16)]
        for b in range(nbuckets):
            stv = st_lo[b] if b < 16 else st_hi[b - 16]
            posblk_v[b] = stv
            fill_v[b] = 0

        dummy_s = jnp.zeros((16,), jnp.int32)
        dummy_1 = jnp.full((16,), dummy1, jnp.int32)
        dummy_2 = jnp.full((16,), dummy2, jnp.int32)

        SW = KBLK + 16

        def flush(b, pos):
            pltpu.sync_copy(stg_s.at[pl.ds(b * SW, KBLK)],
                            o_src.at[pl.ds(pos * KBLK, KBLK)])
            pltpu.sync_copy(stg_1.at[pl.ds(b * SW, KBLK)],
                            o_d1.at[pl.ds(pos * KBLK, KBLK)])
            pltpu.sync_copy(stg_2.at[pl.ds(b * SW, KBLK)],
                            o_d2.at[pl.ds(pos * KBLK, KBLK)])

        def load_body(c5, carry):
            off = wid * per_w + c5 * cblk
            pltpu.sync_copy(keys.at[pl.ds(off, cblk)], kbuf)
            if not payload_iota:
                pltpu.sync_copy(payload.at[pl.ds(off, cblk)], pbuf)

            def vec_body(k, carry2):
                dv = kbuf[pl.ds(k * 16, 16)]
                if payload_iota:
                    pv = lax.iota(jnp.int32, 16) + (off + k * 16)
                else:
                    pv = pbuf[pl.ds(k * 16, 16)]
                cid = lax.shift_right_logical(dv, 13)
                l1 = dv - lax.shift_left(cid, 13)
                l2 = jnp.where(cid >= shift2_thresh,
                               dv - shift2_thresh * CH, dv)
                lane = lax.iota(jnp.int32, 16)
                for b in range(nbuckets):
                    msk = cid == b
                    cntv = jnp.where(msk, 1, 0)
                    for sh in (8, 4, 2, 1):
                        cntv = cntv + cntv.at[lane ^ sh].get(
                            mode="promise_in_bounds")
                    cnt = cntv[0]

                    @pl.when(cnt > 0)
                    def _():
                        fill = fill_v[b]
                        o = b * SW + fill
                        plsc.store_compressed(stg_s.at[pl.ds(o, 16)], pv, mask=msk)
                        plsc.store_compressed(stg_1.at[pl.ds(o, 16)], l1, mask=msk)
                        plsc.store_compressed(stg_2.at[pl.ds(o, 16)], l2, mask=msk)
                        nf = fill + cnt

                        @pl.when(nf >= KBLK)
                        def _():
                            flush(b, posblk_v[b])
                            posblk_v[b] = posblk_v[b] + 1
                            tail_s = stg_s[pl.ds(b * SW + KBLK, 16)]
                            tail_1 = stg_1[pl.ds(b * SW + KBLK, 16)]
                            tail_2 = stg_2[pl.ds(b * SW + KBLK, 16)]
                            stg_s[pl.ds(b * SW, 16)] = tail_s
                            stg_1[pl.ds(b * SW, 16)] = tail_1
                            stg_2[pl.ds(b * SW, 16)] = tail_2

                        fill_v[b] = lax.rem(nf, jnp.int32(KBLK))
                return carry2

            return lax.fori_loop(0, cblk // 16, vec_body, carry)

        lax.fori_loop(0, nload, load_body, 0)

        for b in range(nbuckets):
            fill = fill_v[b]

            @pl.when(fill > 0)
            def _():
                def pad_body(j, carry3):
                    pos = fill + j * 16

                    @pl.when(pos < KBLK)
                    def _():
                        stg_s[pl.ds(b * SW + pos, 16)] = dummy_s
                        stg_1[pl.ds(b * SW + pos, 16)] = dummy_1
                        stg_2[pl.ds(b * SW + pos, 16)] = dummy_2

                    return carry3

                lax.fori_loop(0, 8, pad_body, 0)
                flush(b, posblk_v[b])

    def run(keys, payload, starts_w):
        return pl.kernel(
            body,
            out_type=[jax.ShapeDtypeStruct((ncapb * KBLK,), jnp.int32)] * 3,
            mesh=_sc_mesh(),
            compiler_params=pltpu.CompilerParams(needs_layout_passes=False),
            scratch_types=[
                pltpu.VMEM((32,), jnp.int32),
                pltpu.SMEM((32,), jnp.int32),
                pltpu.SMEM((32,), jnp.int32),
                pltpu.VMEM((nbuckets * (KBLK + 16),), jnp.int32),
                pltpu.VMEM((nbuckets * (KBLK + 16),), jnp.int32),
                pltpu.VMEM((nbuckets * (KBLK + 16),), jnp.int32),
                pltpu.VMEM((cblk,), jnp.int32),
                pltpu.VMEM((cblk,), jnp.int32),
            ],
        )(keys, payload, starts_w)

    return run


_fill_lg = _make_fill(L, NCHUNK, False, CH, CPC * CH, CPC, 2000)
_fill_e = _make_fill(EPAD, 2, True, CH, CH, 1, 2560)


def _make_msg(cpc, out_rows):
    def body(table, init, meta_nb, meta_st, bsrc_b, bldst_b, out,
             acc, mrow, idx_v, ldst_v, rows_v, sem):
        core = lax.axis_index("c")
        s = lax.axis_index("s")
        wid = s * 2 + core
        pltpu.sync_copy(meta_nb.at[wid], mrow)
        nb_lo = mrow[pl.ds(0, 16)]
        nb_hi = mrow[pl.ds(16, 16)]
        pltpu.sync_copy(meta_st.at[wid], mrow)
        st_lo = mrow[pl.ds(0, 16)]
        st_hi = mrow[pl.ds(16, 16)]

        for cb in range(cpc):
            b = core * cpc + cb
            row0 = b * CH + s * RPS
            pltpu.sync_copy(init.at[pl.ds(row0, RPS)],
                            acc.at[pl.ds(s * RPS, RPS)])
            plsc.subcore_barrier()
            for tt in range(2):
                k = cb * 2 + tt
                nblk = nb_lo[k] if k < 16 else nb_hi[k - 16]
                st = st_lo[k] if k < 16 else st_hi[k - 16]

                def blk_body(j, carry2, st=st):
                    posrow = st + j
                    pltpu.sync_copy(bsrc_b.at[pl.ds(posrow * KBLK, KBLK)], idx_v)
                    pltpu.sync_copy(bldst_b.at[pl.ds(posrow * KBLK, KBLK)], ldst_v)
                    pltpu.async_copy(table.at[idx_v], rows_v, sem).wait()
                    pltpu.sync_copy(rows_v, acc.at[ldst_v], add=True)
                    return carry2

                lax.fori_loop(0, nblk, blk_body, 0)
            plsc.subcore_barrier()
            pltpu.sync_copy(acc.at[pl.ds(s * RPS, RPS)],
                            out.at[pl.ds(row0, RPS)])
            plsc.subcore_barrier()

    def run(table, init, meta_nb, meta_st, bsrc_b, bldst_b):
        return pl.kernel(
            body,
            out_type=jax.ShapeDtypeStruct((out_rows, D), jnp.float32),
            mesh=_sc_mesh(),
            scratch_types=[
                pltpu.VMEM_SHARED((CH + 8, D), jnp.float32),
                pltpu.VMEM((32,), jnp.int32),
                pltpu.VMEM((KBLK,), jnp.int32),
                pltpu.VMEM((KBLK,), jnp.int32),
                pltpu.VMEM((KBLK, D), jnp.float32),
                pltpu.SemaphoreType.DMA,
            ],
        )(table, init, meta_nb, meta_st, bsrc_b, bldst_b)

    return run


_msg_lg = _make_msg(CPC, EPAD)
_msg_fin = _make_msg(1, 2 * CH)


def _route_meta(counts, nbuckets, cpc):
    caps = (counts + 127) // KBLK
    capsT = caps.T[:nbuckets]
    flat = capsT.reshape(-1)
    starts_flat = jnp.cumsum(flat) - flat
    starts_bw = starts_flat.reshape(nbuckets, 32)
    starts_w = jnp.zeros((32, 32), jnp.int32).at[:, :nbuckets].set(
        starts_bw.T.astype(jnp.int32))
    w = jnp.arange(32)
    kk = jnp.arange(2 * cpc)
    cb = kk // 2
    tt = kk % 2
    t_idx = 2 * (w[:, None] // 2) + tt[None, :]
    b_idx = (w[:, None] % 2) * cpc + cb[None, :]
    meta_nb = jnp.zeros((32, 32), jnp.int32).at[:, :2 * cpc].set(
        caps[t_idx, b_idx].astype(jnp.int32))
    meta_st = jnp.zeros((32, 32), jnp.int32).at[:, :2 * cpc].set(
        starts_bw[b_idx, t_idx].astype(jnp.int32))
    return starts_w, meta_nb, meta_st


PBLK = 2000
NPB = E // PBLK
NEG = -3.0e38


def _matvec_body(o_ref, w_ref, rs_ref):
    rs_ref[...] = jnp.dot(o_ref[...], w_ref[...],
                          preferred_element_type=jnp.float32)


def _matvec(out_pad, Wcat):
    return pl.pallas_call(
        _matvec_body,
        grid=(EPAD // 2048,),
        in_specs=[
            pl.BlockSpec((2048, D), lambda i: (i, 0)),
            pl.BlockSpec((D, 16), lambda i: (0, 0)),
        ],
        out_specs=pl.BlockSpec((2048, 16), lambda i: (i, 0)),
        out_shape=jax.ShapeDtypeStruct((EPAD, 16), jnp.float32),
    )(out_pad, Wcat)


def _xc_oh(rs_ref, rs1_ref, rse_ref, b_ref):
    xc = rs_ref[:, 0] + rs1_ref[:, 1] - rse_ref[:, 1]
    bb = b_ref[0, 0, :]
    oh = (bb[:, None]
          == jax.lax.broadcasted_iota(jnp.int32, (PBLK, B), 1))
    return xc, oh


def _pmax_body(rs_ref, rs1_ref, rse_ref, b_ref, m_ref):
    i = pl.program_id(0)
    xc, oh = _xc_oh(rs_ref, rs1_ref, rse_ref, b_ref)

    @pl.when(i == 0)
    def _():
        m_ref[...] = jnp.full((1, B), NEG, jnp.float32)

    mp = jnp.max(jnp.where(oh, xc[:, None], NEG), axis=0)
    m_ref[...] = jnp.maximum(m_ref[...], mp[None, :])


def _pden_body(rs_ref, rs1_ref, rse_ref, b_ref, m_ref, den_ref):
    i = pl.program_id(0)
    xc, oh = _xc_oh(rs_ref, rs1_ref, rse_ref, b_ref)

    @pl.when(i == 0)
    def _():
        den_ref[...] = jnp.zeros((1, B), jnp.float32)

    mb = jnp.max(jnp.where(oh, m_ref[...], NEG), axis=1)
    ex = jnp.exp(xc - mb)
    dp = jnp.sum(jnp.where(oh, ex[:, None], 0.0), axis=0)
    den_ref[...] = den_ref[...] + dp[None, :]


def _pgx_body(rs_ref, rs1_ref, rse_ref, b_ref, m_ref, den_ref, o_ref,
              wg_ref, bg_ref, gx_ref, gout_ref):
    i = pl.program_id(0)
    xc, oh = _xc_oh(rs_ref, rs1_ref, rse_ref, b_ref)

    @pl.when(i == 0)
    def _():
        gx_ref[...] = jnp.zeros((B, D), jnp.float32)

    mb = jnp.max(jnp.where(oh, m_ref[...], NEG), axis=1)
    db = jnp.sum(jnp.where(oh, den_ref[...], 0.0), axis=1)
    sc = jnp.exp(xc - mb) / db
    w = o_ref[...] * sc[:, None]
    gxp = jax.lax.dot_general(oh.astype(jnp.float32), w,
                              (((0,), (0,)), ((), ())),
                              preferred_element_type=jnp.float32)
    gx_ref[...] = gx_ref[...] + gxp

    @pl.when(i == NPB - 1)
    def _():
        gout_ref[...] = jnp.tanh(
            jnp.dot(gx_ref[...], wg_ref[...],
                    preferred_element_type=jnp.float32) + bg_ref[...])


def _pool(rs_n, rs_n1, rs_e0, batch3d, out_pad, W_gout, b_gout):
    rspec = pl.BlockSpec((PBLK, 16), lambda i: (i, 0))
    bspec = pl.BlockSpec((1, 1, PBLK), lambda i: (i, 0, 0))
    full = pl.BlockSpec((1, B), lambda i: (0, 0))
    m = pl.pallas_call(
        _pmax_body,
        grid=(NPB,),
        in_specs=[rspec, rspec, rspec, bspec],
        out_specs=full,
        out_shape=jax.ShapeDtypeStruct((1, B), jnp.float32),
    )(rs_n, rs_n1, rs_e0, batch3d)
    den = pl.pallas_call(
        _pden_body,
        grid=(NPB,),
        in_specs=[rspec, rspec, rspec, bspec, full],
        out_specs=full,
        out_shape=jax.ShapeDtypeStruct((1, B), jnp.float32),
    )(rs_n, rs_n1, rs_e0, batch3d, m)
    _, gout = pl.pallas_call(
        _pgx_body,
        grid=(NPB,),
        in_specs=[rspec, rspec, rspec, bspec, full, full,
                  pl.BlockSpec((PBLK, D), lambda i: (i, 0)),
                  pl.BlockSpec((D, D), lambda i: (0, 0)),
                  pl.BlockSpec((1, D), lambda i: (0, 0))],
        out_specs=[pl.BlockSpec((B, D), lambda i: (0, 0))] * 2,
        out_shape=[jax.ShapeDtypeStruct((B, D), jnp.float32)] * 2,
    )(rs_n, rs_n1, rs_e0, batch3d, m, den, out_pad, W_gout,
      b_gout.reshape(1, D))
    return gout


def _scores_body(g0, g1, g2, g3, a_ref, ab_ref, s_ref):
    cols = []
    for n, g in enumerate((g0, g1, g2, g3)):
        sn = jnp.sum(g[...] * a_ref[:, n][None, :], axis=1) + ab_ref[0, n]
        cols.append(sn[:, None])
    S = jnp.concatenate(cols, axis=1)
    mx = jnp.max(S, axis=1, keepdims=True)
    ex = jnp.exp(S - mx)
    P = ex / jnp.sum(ex, axis=1, keepdims=True)
    s_ref[...] = jnp.concatenate(
        [P, jnp.zeros((B, 4), jnp.float32)], axis=1)


def _scores(gouts, a, a_bias):
    gspec = pl.BlockSpec((B, D), lambda: (0, 0))
    return pl.pallas_call(
        _scores_body,
        grid=(),
        in_specs=[gspec, gspec, gspec, gspec,
                  pl.BlockSpec((D, 4), lambda: (0, 0)),
                  pl.BlockSpec((1, 4), lambda: (0, 0))],
        out_specs=pl.BlockSpec((B, 8), lambda: (0, 0)),
        out_shape=jax.ShapeDtypeStruct((B, 8), jnp.float32),
    )(*gouts, a.reshape(D, NITER), a_bias.reshape(1, NITER))


def _fin_body(o0, o1, o2, o3, b_ref, s_ref, of_ref):
    bb = b_ref[0, 0, :]
    oh = (bb[:, None]
          == jax.lax.broadcasted_iota(jnp.int32, (PBLK, B), 1))
    se = jax.lax.dot_general(oh.astype(jnp.float32), s_ref[...],
                             (((1,), (0,)), ((), ())),
                             preferred_element_type=jnp.float32)
    acc = o0[...] * se[:, 0][:, None]
    for n, o in enumerate((o1, o2, o3)):
        acc = acc + o[...] * se[:, n + 1][:, None]
    of_ref[...] = acc


def _fin(outs, batch3d, scores):
    ospec = pl.BlockSpec((PBLK, D), lambda i: (i, 0))
    return pl.pallas_call(
        _fin_body,
        grid=(NPB,),
        in_specs=[ospec, ospec, ospec, ospec,
                  pl.BlockSpec((1, 1, PBLK), lambda i: (i, 0, 0)),
                  pl.BlockSpec((B, 8), lambda i: (0, 0))],
        out_specs=ospec,
        out_shape=jax.ShapeDtypeStruct((EPAD, D), jnp.float32),
    )(*outs, batch3d, scores)


def kernel(x, edge_attr, edge_index, line_graph_edge_index, edge_index_batch,
           W_mlp, b_mlp, W_u, W_v, W_edge, W_att_root, W_att_rel, b_att, a,
           W_gout, b_gout, a_bias, W_lb, b_lb):
    h, eu3, ev3 = _pre_node(x, W_mlp, b_mlp, W_u, W_v)
    euv3 = _pre_edge(edge_attr, W_edge)

    src = edge_index[0].astype(jnp.int32)
    dst = edge_index[1].astype(jnp.int32)
    lg_src = line_graph_edge_index[0].astype(jnp.int32)
    lg_dst = line_graph_edge_index[1].astype(jnp.int32)
    batch = edge_index_batch.astype(jnp.int32)

    table = jnp.concatenate([eu3, ev3], axis=0)
    src_pad = jnp.pad(src, (0, EPAD - E))
    dst_pad = jnp.pad(dst, (0, EPAD - E))
    gidx_blocks = jnp.concatenate([src_pad, dst_pad + N]).reshape(-1, KBLK)
    iota_rows = jnp.arange(CH, dtype=jnp.int32).reshape(CH // KBLK, KBLK)

    e0_pad = _sc_e0(table, gidx_blocks, iota_rows, euv3)

    counts_lg = _count_lg(lg_dst)
    starts_w, meta_nb, meta_st = _route_meta(counts_lg, NCHUNK, CPC)
    bsrc_b, bldst_b, bldst2_b = _fill_lg(lg_dst, lg_src, starts_w)

    outs = [e0_pad]
    for k in range(NITER + 1):
        outs.append(_msg_lg(outs[-1], e0_pad, meta_nb, meta_st,
                            bsrc_b, bldst_b))

    Wcat = jnp.concatenate(
        [W_att_root, W_att_rel, jnp.zeros((D, 14), jnp.float32)], axis=1)
    rs = [_matvec(o, Wcat) for o in outs]

    batch3d = batch.reshape(NPB, 1, PBLK)
    gouts = [
        _pool(rs[n], rs[n + 1], rs[0], batch3d, outs[n], W_gout, b_gout)
        for n in range(1, NITER + 1)
    ]
    scores = _scores(gouts, a, a_bias)
    out_fin = _fin(outs[1:NITER + 1], batch3d, scores)

    counts_e = _count_e(jnp.pad(dst, (0, EPAD - E), constant_values=2 * CH))
    starts_we, meta_nbe, meta_ste = _route_meta(counts_e, 2, 1)
    bsrc_e, bldst_e, _ = _fill_e(
        jnp.pad(dst, (0, EPAD - E), constant_values=2 * CH), dst_pad,
        starts_we)
    h_pad = jnp.pad(h, ((0, 2 * CH - N), (0, 0)))
    xo_pad = _msg_fin(out_fin, h_pad, meta_nbe, meta_ste, bsrc_e, bldst_e)
    return _final_linear(xo_pad, W_lb, b_lb)


# speedup vs baseline: 1.9380x; 1.0074x over previous
import functools

import jax
import jax.lax as lax
import jax.numpy as jnp
from jax.experimental import pallas as pl
from jax.experimental.pallas import tpu as pltpu
from jax.experimental.pallas import tpu_sc as plsc

N = 10000
E = 160000
L = 320000
B = 256
D = 128
ED = 6
NITER = 4

CH = 8192
NCHUNK = 20
EPAD = CH * NCHUNK
CPC = NCHUNK // 2
KBLK = 128
NSUB = 16
RPS = CH // NSUB
BPS = RPS // KBLK

@functools.cache
def _sc_mesh():
    return plsc.VectorSubcoreMesh(core_axis_name="c", subcore_axis_name="s",
                                  num_cores=2, num_subcores=NSUB)


def _pre_node_body(x_ref, wmlp_ref, bmlp_ref, wu_ref, wv_ref,
                   h_ref, eu_ref, ev_ref):
    h = jnp.dot(x_ref[...], wmlp_ref[...],
                preferred_element_type=jnp.float32) + bmlp_ref[...]
    h_ref[...] = h
    third = jnp.float32(1.0 / 3.0)
    eu_ref[...] = jnp.dot(h, wu_ref[...],
                          preferred_element_type=jnp.float32) * third
    ev_ref[...] = jnp.dot(h, wv_ref[...],
                          preferred_element_type=jnp.float32) * third


def _pre_node(x, W_mlp, b_mlp, W_u, W_v):
    blk = 1000
    return pl.pallas_call(
        _pre_node_body,
        grid=(N // blk,),
        in_specs=[
            pl.BlockSpec((blk, D), lambda i: (i, 0)),
            pl.BlockSpec((D, D), lambda i: (0, 0)),
            pl.BlockSpec((1, D), lambda i: (0, 0)),
            pl.BlockSpec((D, D), lambda i: (0, 0)),
            pl.BlockSpec((D, D), lambda i: (0, 0)),
        ],
        out_specs=[
            pl.BlockSpec((blk, D), lambda i: (i, 0)),
            pl.BlockSpec((blk, D), lambda i: (i, 0)),
            pl.BlockSpec((blk, D), lambda i: (i, 0)),
        ],
        out_shape=[jax.ShapeDtypeStruct((N, D), jnp.float32)] * 3,
    )(x, W_mlp, b_mlp.reshape(1, D), W_u, W_v)


def _pre_edge_body(ea_ref, we_ref, euv_ref):
    euv_ref[...] = jnp.dot(ea_ref[...], we_ref[...],
                           preferred_element_type=jnp.float32) * jnp.float32(1.0 / 3.0)


def _pre_edge(edge_attr, W_edge):
    blk = 2048
    ea = jnp.pad(edge_attr, ((0, EPAD - E), (0, 8 - ED)))
    we = jnp.pad(W_edge, ((0, 8 - ED), (0, 0)))
    return pl.pallas_call(
        _pre_edge_body,
        grid=(EPAD // blk,),
        in_specs=[
            pl.BlockSpec((blk, 8), lambda i: (i, 0)),
            pl.BlockSpec((8, D), lambda i: (0, 0)),
        ],
        out_specs=pl.BlockSpec((blk, D), lambda i: (i, 0)),
        out_shape=jax.ShapeDtypeStruct((EPAD, D), jnp.float32),
    )(ea, we)


def _final_body(xo_ref, w_ref, b_ref, o_ref):
    o_ref[...] = jnp.dot(xo_ref[...], w_ref[...],
                         preferred_element_type=jnp.float32) + b_ref[...]


def _final_linear(xo_pad, W_lb, b_lb):
    blk = 1000
    return pl.pallas_call(
        _final_body,
        grid=(N // blk,),
        in_specs=[
            pl.BlockSpec((blk, D), lambda i: (i, 0)),
            pl.BlockSpec((D, D), lambda i: (0, 0)),
            pl.BlockSpec((1, D), lambda i: (0, 0)),
        ],
        out_specs=pl.BlockSpec((blk, D), lambda i: (i, 0)),
        out_shape=jax.ShapeDtypeStruct((N, D), jnp.float32),
    )(xo_pad, W_lb, b_lb.reshape(1, D))


def _e0_body(table, gidx_blocks, iota_rows, init, out,
             acc, idx_v, ldst_v, rows_v, sem):
    core = lax.axis_index("c")
    s = lax.axis_index("s")
    pltpu.sync_copy(iota_rows.at[pl.ds(s * BPS, BPS)], ldst_v)

    def chunk_body(cb, carry):
        b = core * CPC + cb
        row0 = b * CH + s * RPS
        pltpu.sync_copy(init.at[pl.ds(row0, RPS)], acc.at[pl.ds(s * RPS, RPS)])
        plsc.subcore_barrier()

        def part_body(p, carry2):
            def blk_body(jj, carry3):
                blkrow = p * (EPAD // KBLK) + b * (CH // KBLK) + s * BPS + jj
                pltpu.sync_copy(gidx_blocks.at[blkrow], idx_v)
                pltpu.async_copy(table.at[idx_v], rows_v, sem).wait()
                pltpu.sync_copy(rows_v, acc.at[ldst_v.at[jj]], add=True)
                return carry3
            return lax.fori_loop(0, BPS, blk_body, carry2)

        lax.fori_loop(0, 2, part_body, 0)
        plsc.subcore_barrier()
        pltpu.sync_copy(acc.at[pl.ds(s * RPS, RPS)], out.at[pl.ds(row0, RPS)])
        plsc.subcore_barrier()
        return carry

    lax.fori_loop(0, CPC, chunk_body, 0)


@functools.partial(jax.jit, static_argnames=())
def _sc_e0(table, gidx_blocks, iota_rows, init):
    return pl.kernel(
        _e0_body,
        out_type=jax.ShapeDtypeStruct((EPAD, D), jnp.float32),
        mesh=_sc_mesh(),
        scratch_types=[
            pltpu.VMEM_SHARED((CH, D), jnp.float32),
            pltpu.VMEM((KBLK,), jnp.int32),
            pltpu.VMEM((BPS, KBLK), jnp.int32),
            pltpu.VMEM((KBLK, D), jnp.float32),
            pltpu.SemaphoreType.DMA,
        ],
    )(table, gidx_blocks, iota_rows, init)


def _make_count(total, nbuckets, cblk):
    per_w = total // 32
    nload = per_w // cblk

    def body(keys, out, cnt_v, buf):
        core = lax.axis_index("c")
        s = lax.axis_index("s")
        wid = s * 2 + core
        zeros16 = jnp.zeros((16,), jnp.int32)
        for b in range(nbuckets):
            cnt_v[pl.ds(b * 16, 16)] = zeros16

        def load_body(c5, carry):
            pltpu.sync_copy(keys.at[pl.ds(wid * per_w + c5 * cblk, cblk)], buf)

            def vec_body(k, carry2):
                dv = buf[pl.ds(k * 16, 16)]
                cid = lax.shift_right_logical(dv, 13)
                for b in range(nbuckets):
                    cnt_v[pl.ds(b * 16, 16)] = (
                        cnt_v[pl.ds(b * 16, 16)]
                        + jnp.where(cid == b, 1, 0))
                return carry2

            return lax.fori_loop(0, cblk // 16, vec_body, carry)

        lax.fori_loop(0, nload, load_body, 0)
        pltpu.sync_copy(cnt_v, out.at[wid])

    def run(keys):
        lanes = pl.kernel(
            body,
            out_type=jax.ShapeDtypeStruct((32, nbuckets * 16), jnp.int32),
            mesh=_sc_mesh(),
            scratch_types=[
                pltpu.VMEM((nbuckets * 16,), jnp.int32),
                pltpu.VMEM((cblk,), jnp.int32),
            ],
        )(keys)
        counts = lanes.reshape(32, nbuckets, 16).sum(-1)
        return jnp.zeros((32, 32), jnp.int32).at[:, :nbuckets].set(counts)

    return run


_count_lg = _make_count(L, NCHUNK, 2000)
_count_e = _make_count(EPAD, 2, 2560)


def _make_fill(total, nbuckets, payload_iota, dummy1, dummy2, shift2_thresh, cblk):
    per_w = total // 32
    nload = per_w // cblk
    ncapb = total // KBLK + 32 * nbuckets

    def body(keys, payload, starts_w, o_src, o_d1, o_d2,
             rowbuf, posblk_v, fill_v,
             stg_s, stg_1, stg_2, kbuf, pbuf):
        core = lax.axis_index("c")
        s = lax.axis_index("s")
        wid = s * 2 + core
        pltpu.sync_copy(starts_w.at[wid], rowbuf)
        st_lo = rowbuf[pl.ds(0, 16)]
        st_hi = rowbuf[pl.ds(16, 16)]
        for b in range(nbuckets):
            stv = st_lo[b] if b < 16 else st_hi[b - 16]
            posblk_v[b] = stv
            fill_v[b] = 0

        dummy_s = jnp.zeros((16,), jnp.int32)
        dummy_1 = jnp.full((16,), dummy1, jnp.int32)
        dummy_2 = jnp.full((16,), dummy2, jnp.int32)

        SW = KBLK + 16

        def flush(b, pos):
            pltpu.sync_copy(stg_s.at[pl.ds(b * SW, KBLK)],
                            o_src.at[pl.ds(pos * KBLK, KBLK)])
            pltpu.sync_copy(stg_1.at[pl.ds(b * SW, KBLK)],
                            o_d1.at[pl.ds(pos * KBLK, KBLK)])
            pltpu.sync_copy(stg_2.at[pl.ds(b * SW, KBLK)],
                            o_d2.at[pl.ds(pos * KBLK, KBLK)])

        def load_body(c5, carry):
            off = wid * per_w + c5 * cblk
            pltpu.sync_copy(keys.at[pl.ds(off, cblk)], kbuf)
            if not payload_iota:
                pltpu.sync_copy(payload.at[pl.ds(off, cblk)], pbuf)

            def vec_body(k, carry2):
                dv = kbuf[pl.ds(k * 16, 16)]
                if payload_iota:
                    pv = lax.iota(jnp.int32, 16) + (off + k * 16)
                else:
                    pv = pbuf[pl.ds(k * 16, 16)]
                cid = lax.shift_right_logical(dv, 13)
                l1 = dv - lax.shift_left(cid, 13)
                l2 = jnp.where(cid >= shift2_thresh,
                               dv - shift2_thresh * CH, dv)
                lane = lax.iota(jnp.int32, 16)
                for b in range(nbuckets):
                    msk = cid == b
                    cntv = jnp.where(msk, 1, 0)
                    for sh in (8, 4, 2, 1):
                        cntv = cntv + cntv.at[lane ^ sh].get(
                            mode="promise_in_bounds")
                    cnt = cntv[0]

                    @pl.when(cnt > 0)
                    def _():
                        fill = fill_v[b]
                        o = b * SW + fill
                        plsc.store_compressed(stg_s.at[pl.ds(o, 16)], pv, mask=msk)
                        plsc.store_compressed(stg_1.at[pl.ds(o, 16)], l1, mask=msk)
                        plsc.store_compressed(stg_2.at[pl.ds(o, 16)], l2, mask=msk)
                        nf = fill + cnt

                        @pl.when(nf >= KBLK)
                        def _():
                            flush(b, posblk_v[b])
                            posblk_v[b] = posblk_v[b] + 1
                            tail_s = stg_s[pl.ds(b * SW + KBLK, 16)]
                            tail_1 = stg_1[pl.ds(b * SW + KBLK, 16)]
                            tail_2 = stg_2[pl.ds(b * SW + KBLK, 16)]
                            stg_s[pl.ds(b * SW, 16)] = tail_s
                            stg_1[pl.ds(b * SW, 16)] = tail_1
                            stg_2[pl.ds(b * SW, 16)] = tail_2

                        fill_v[b] = lax.rem(nf, jnp.int32(KBLK))
                return carry2

            return lax.fori_loop(0, cblk // 16, vec_body, carry)

        lax.fori_loop(0, nload, load_body, 0)

        for b in range(nbuckets):
            fill = fill_v[b]

            @pl.when(fill > 0)
            def _():
                def pad_body(j, carry3):
                    pos = fill + j * 16

                    @pl.when(pos < KBLK)
                    def _():
                        stg_s[pl.ds(b * SW + pos, 16)] = dummy_s
                        stg_1[pl.ds(b * SW + pos, 16)] = dummy_1
                        stg_2[pl.ds(b * SW + pos, 16)] = dummy_2

                    return carry3

                lax.fori_loop(0, 8, pad_body, 0)
                flush(b, posblk_v[b])

    def run(keys, payload, starts_w):
        return pl.kernel(
            body,
            out_type=[jax.ShapeDtypeStruct((ncapb * KBLK,), jnp.int32)] * 3,
            mesh=_sc_mesh(),
            compiler_params=pltpu.CompilerParams(needs_layout_passes=False),
            scratch_types=[
                pltpu.VMEM((32,), jnp.int32),
                pltpu.SMEM((32,), jnp.int32),
                pltpu.SMEM((32,), jnp.int32),
                pltpu.VMEM((nbuckets * (KBLK + 16),), jnp.int32),
                pltpu.VMEM((nbuckets * (KBLK + 16),), jnp.int32),
                pltpu.VMEM((nbuckets * (KBLK + 16),), jnp.int32),
                pltpu.VMEM((cblk,), jnp.int32),
                pltpu.VMEM((cblk,), jnp.int32),
            ],
        )(keys, payload, starts_w)

    return run


_fill_lg = _make_fill(L, NCHUNK, False, CH, CPC * CH, CPC, 2000)
_fill_e = _make_fill(EPAD, 2, True, CH, CH, 1, 2560)


def _make_msg(cpc, out_rows):
    def body(table, init, meta_nb, meta_st, bsrc_b, bldst_b, out,
             acc, mrow, idx_v0, idx_v1, ldst_v0, ldst_v1, rows_v0, rows_v1,
             gsem, ssem0, ssem1):
        core = lax.axis_index("c")
        s = lax.axis_index("s")
        wid = s * 2 + core
        pltpu.sync_copy(meta_nb.at[wid], mrow)
        nb_lo = mrow[pl.ds(0, 16)]
        nb_hi = mrow[pl.ds(16, 16)]
        pltpu.sync_copy(meta_st.at[wid], mrow)
        st_lo = mrow[pl.ds(0, 16)]
        st_hi = mrow[pl.ds(16, 16)]
        for cb in range(cpc):
            b = core * cpc + cb
            row0 = b * CH + s * RPS
            pltpu.sync_copy(init.at[pl.ds(row0, RPS)],
                            acc.at[pl.ds(s * RPS, RPS)])
            plsc.subcore_barrier()
            for tt in range(2):
                k = cb * 2 + tt
                nblk = nb_lo[k] if k < 16 else nb_hi[k - 16]
                st = st_lo[k] if k < 16 else st_hi[k - 16]

                def pair_body(jj, carry, st=st):
                    j0 = st + jj * 2
                    pltpu.sync_copy(bsrc_b.at[pl.ds(j0 * KBLK, KBLK)], idx_v0)
                    pltpu.sync_copy(bldst_b.at[pl.ds(j0 * KBLK, KBLK)],
                                    ldst_v0)
                    g0 = pltpu.async_copy(table.at[idx_v0], rows_v0, gsem)
                    pltpu.sync_copy(bsrc_b.at[pl.ds((j0 + 1) * KBLK, KBLK)],
                                    idx_v1)
                    pltpu.sync_copy(bldst_b.at[pl.ds((j0 + 1) * KBLK, KBLK)],
                                    ldst_v1)
                    g0.wait()
                    s0 = pltpu.async_copy(rows_v0, acc.at[ldst_v0], ssem0,
                                          add=True)
                    g1 = pltpu.async_copy(table.at[idx_v1], rows_v1, gsem)
                    g1.wait()
                    s1 = pltpu.async_copy(rows_v1, acc.at[ldst_v1], ssem1,
                                          add=True)
                    s0.wait()
                    s1.wait()
                    return carry

                lax.fori_loop(0, nblk // 2, pair_body, 0)

                @pl.when(lax.rem(nblk, 2) == 1)
                def _(st=st, nblk=nblk):
                    j = st + nblk - 1
                    pltpu.sync_copy(bsrc_b.at[pl.ds(j * KBLK, KBLK)], idx_v0)
                    pltpu.sync_copy(bldst_b.at[pl.ds(j * KBLK, KBLK)], ldst_v0)
                    pltpu.async_copy(table.at[idx_v0], rows_v0, gsem).wait()
                    pltpu.sync_copy(rows_v0, acc.at[ldst_v0], add=True)
            plsc.subcore_barrier()
            pltpu.sync_copy(acc.at[pl.ds(s * RPS, RPS)],
                            out.at[pl.ds(row0, RPS)])
            plsc.subcore_barrier()

    def run(table, init, meta_nb, meta_st, bsrc_b, bldst_b):
        return pl.kernel(
            body,
            out_type=jax.ShapeDtypeStruct((out_rows, D), jnp.float32),
            mesh=_sc_mesh(),
            scratch_types=[
                pltpu.VMEM_SHARED((CH + 8, D), jnp.float32),
                pltpu.VMEM((32,), jnp.int32),
                pltpu.VMEM((KBLK,), jnp.int32),
                pltpu.VMEM((KBLK,), jnp.int32),
                pltpu.VMEM((KBLK,), jnp.int32),
                pltpu.VMEM((KBLK,), jnp.int32),
                pltpu.VMEM((KBLK, D), jnp.float32),
                pltpu.VMEM((KBLK, D), jnp.float32),
                pltpu.SemaphoreType.DMA,
                pltpu.SemaphoreType.DMA,
                pltpu.SemaphoreType.DMA,
            ],
        )(table, init, meta_nb, meta_st, bsrc_b, bldst_b)

    return run


_msg_lg = _make_msg(CPC, EPAD)
_msg_fin = _make_msg(1, 2 * CH)


def _route_meta(counts, nbuckets, cpc):
    caps = (counts + 127) // KBLK
    capsT = caps.T[:nbuckets]
    flat = capsT.reshape(-1)
    starts_flat = jnp.cumsum(flat) - flat
    starts_bw = starts_flat.reshape(nbuckets, 32)
    starts_w = jnp.zeros((32, 32), jnp.int32).at[:, :nbuckets].set(
        starts_bw.T.astype(jnp.int32))
    w = jnp.arange(32)
    kk = jnp.arange(2 * cpc)
    cb = kk // 2
    tt = kk % 2
    t_idx = 2 * (w[:, None] // 2) + tt[None, :]
    b_idx = (w[:, None] % 2) * cpc + cb[None, :]
    meta_nb = jnp.zeros((32, 32), jnp.int32).at[:, :2 * cpc].set(
        caps[t_idx, b_idx].astype(jnp.int32))
    meta_st = jnp.zeros((32, 32), jnp.int32).at[:, :2 * cpc].set(
        starts_bw[b_idx, t_idx].astype(jnp.int32))
    return starts_w, meta_nb, meta_st


PBLK = 2000
NPB = E // PBLK
NEG = -3.0e38


def _matvec_body(o_ref, w_ref, rs_ref):
    rs_ref[...] = jnp.dot(o_ref[...], w_ref[...],
                          preferred_element_type=jnp.float32)


def _matvec(out_pad, Wcat):
    return pl.pallas_call(
        _matvec_body,
        grid=(EPAD // 2048,),
        in_specs=[
            pl.BlockSpec((2048, D), lambda i: (i, 0)),
            pl.BlockSpec((D, 16), lambda i: (0, 0)),
        ],
        out_specs=pl.BlockSpec((2048, 16), lambda i: (i, 0)),
        out_shape=jax.ShapeDtypeStruct((EPAD, 16), jnp.float32),
    )(out_pad, Wcat)


def _xc_oh(rs_ref, rs1_ref, rse_ref, b_ref):
    xc = rs_ref[:, 0] + rs1_ref[:, 1] - rse_ref[:, 1]
    bb = b_ref[0, 0, :]
    oh = (bb[:, None]
          == jax.lax.broadcasted_iota(jnp.int32, (PBLK, B), 1))
    return xc, oh


def _pmax_body(rs_ref, rs1_ref, rse_ref, b_ref, m_ref):
    i = pl.program_id(0)
    xc, oh = _xc_oh(rs_ref, rs1_ref, rse_ref, b_ref)

    @pl.when(i == 0)
    def _():
        m_ref[...] = jnp.full((1, B), NEG, jnp.float32)

    mp = jnp.max(jnp.where(oh, xc[:, None], NEG), axis=0)
    m_ref[...] = jnp.maximum(m_ref[...], mp[None, :])


def _pden_body(rs_ref, rs1_ref, rse_ref, b_ref, m_ref, den_ref):
    i = pl.program_id(0)
    xc, oh = _xc_oh(rs_ref, rs1_ref, rse_ref, b_ref)

    @pl.when(i == 0)
    def _():
        den_ref[...] = jnp.zeros((1, B), jnp.float32)

    mb = jnp.max(jnp.where(oh, m_ref[...], NEG), axis=1)
    ex = jnp.exp(xc - mb)
    dp = jnp.sum(jnp.where(oh, ex[:, None], 0.0), axis=0)
    den_ref[...] = den_ref[...] + dp[None, :]


def _pgx_body(rs_ref, rs1_ref, rse_ref, b_ref, m_ref, den_ref, o_ref,
              wg_ref, bg_ref, gx_ref, gout_ref):
    i = pl.program_id(0)
    xc, oh = _xc_oh(rs_ref, rs1_ref, rse_ref, b_ref)

    @pl.when(i == 0)
    def _():
        gx_ref[...] = jnp.zeros((B, D), jnp.float32)

    mb = jnp.max(jnp.where(oh, m_ref[...], NEG), axis=1)
    db = jnp.sum(jnp.where(oh, den_ref[...], 0.0), axis=1)
    sc = jnp.exp(xc - mb) / db
    w = o_ref[...] * sc[:, None]
    gxp = jax.lax.dot_general(oh.astype(jnp.float32), w,
                              (((0,), (0,)), ((), ())),
                              preferred_element_type=jnp.float32)
    gx_ref[...] = gx_ref[...] + gxp

    @pl.when(i == NPB - 1)
    def _():
        gout_ref[...] = jnp.tanh(
            jnp.dot(gx_ref[...], wg_ref[...],
                    preferred_element_type=jnp.float32) + bg_ref[...])


def _pool(rs_n, rs_n1, rs_e0, batch3d, out_pad, W_gout, b_gout):
    rspec = pl.BlockSpec((PBLK, 16), lambda i: (i, 0))
    bspec = pl.BlockSpec((1, 1, PBLK), lambda i: (i, 0, 0))
    full = pl.BlockSpec((1, B), lambda i: (0, 0))
    m = pl.pallas_call(
        _pmax_body,
        grid=(NPB,),
        in_specs=[rspec, rspec, rspec, bspec],
        out_specs=full,
        out_shape=jax.ShapeDtypeStruct((1, B), jnp.float32),
    )(rs_n, rs_n1, rs_e0, batch3d)
    den = pl.pallas_call(
        _pden_body,
        grid=(NPB,),
        in_specs=[rspec, rspec, rspec, bspec, full],
        out_specs=full,
        out_shape=jax.ShapeDtypeStruct((1, B), jnp.float32),
    )(rs_n, rs_n1, rs_e0, batch3d, m)
    _, gout = pl.pallas_call(
        _pgx_body,
        grid=(NPB,),
        in_specs=[rspec, rspec, rspec, bspec, full, full,
                  pl.BlockSpec((PBLK, D), lambda i: (i, 0)),
                  pl.BlockSpec((D, D), lambda i: (0, 0)),
                  pl.BlockSpec((1, D), lambda i: (0, 0))],
        out_specs=[pl.BlockSpec((B, D), lambda i: (0, 0))] * 2,
        out_shape=[jax.ShapeDtypeStruct((B, D), jnp.float32)] * 2,
    )(rs_n, rs_n1, rs_e0, batch3d, m, den, out_pad, W_gout,
      b_gout.reshape(1, D))
    return gout


def _scores_body(g0, g1, g2, g3, a_ref, ab_ref, s_ref):
    cols = []
    for n, g in enumerate((g0, g1, g2, g3)):
        sn = jnp.sum(g[...] * a_ref[:, n][None, :], axis=1) + ab_ref[0, n]
        cols.append(sn[:, None])
    S = jnp.concatenate(cols, axis=1)
    mx = jnp.max(S, axis=1, keepdims=True)
    ex = jnp.exp(S - mx)
    P = ex / jnp.sum(ex, axis=1, keepdims=True)
    s_ref[...] = jnp.concatenate(
        [P, jnp.zeros((B, 4), jnp.float32)], axis=1)


def _scores(gouts, a, a_bias):
    gspec = pl.BlockSpec((B, D), lambda: (0, 0))
    return pl.pallas_call(
        _scores_body,
        grid=(),
        in_specs=[gspec, gspec, gspec, gspec,
                  pl.BlockSpec((D, 4), lambda: (0, 0)),
                  pl.BlockSpec((1, 4), lambda: (0, 0))],
        out_specs=pl.BlockSpec((B, 8), lambda: (0, 0)),
        out_shape=jax.ShapeDtypeStruct((B, 8), jnp.float32),
    )(*gouts, a.reshape(D, NITER), a_bias.reshape(1, NITER))


def _fin_body(o0, o1, o2, o3, b_ref, s_ref, of_ref):
    bb = b_ref[0, 0, :]
    oh = (bb[:, None]
          == jax.lax.broadcasted_iota(jnp.int32, (PBLK, B), 1))
    se = jax.lax.dot_general(oh.astype(jnp.float32), s_ref[...],
                             (((1,), (0,)), ((), ())),
                             preferred_element_type=jnp.float32)
    acc = o0[...] * se[:, 0][:, None]
    for n, o in enumerate((o1, o2, o3)):
        acc = acc + o[...] * se[:, n + 1][:, None]
    of_ref[...] = acc


def _fin(outs, batch3d, scores):
    ospec = pl.BlockSpec((PBLK, D), lambda i: (i, 0))
    return pl.pallas_call(
        _fin_body,
        grid=(NPB,),
        in_specs=[ospec, ospec, ospec, ospec,
                  pl.BlockSpec((1, 1, PBLK), lambda i: (i, 0, 0)),
                  pl.BlockSpec((B, 8), lambda i: (0, 0))],
        out_specs=ospec,
        out_shape=jax.ShapeDtypeStruct((EPAD, D), jnp.float32),
    )(*outs, batch3d, scores)


def kernel(x, edge_attr, edge_index, line_graph_edge_index, edge_index_batch,
           W_mlp, b_mlp, W_u, W_v, W_edge, W_att_root, W_att_rel, b_att, a,
           W_gout, b_gout, a_bias, W_lb, b_lb):
    h, eu3, ev3 = _pre_node(x, W_mlp, b_mlp, W_u, W_v)
    euv3 = _pre_edge(edge_attr, W_edge)

    src = edge_index[0].astype(jnp.int32)
    dst = edge_index[1].astype(jnp.int32)
    lg_src = line_graph_edge_index[0].astype(jnp.int32)
    lg_dst = line_graph_edge_index[1].astype(jnp.int32)
    batch = edge_index_batch.astype(jnp.int32)

    table = jnp.concatenate([eu3, ev3], axis=0)
    src_pad = jnp.pad(src, (0, EPAD - E))
    dst_pad = jnp.pad(dst, (0, EPAD - E))
    gidx_blocks = jnp.concatenate([src_pad, dst_pad + N]).reshape(-1, KBLK)
    iota_rows = jnp.arange(CH, dtype=jnp.int32).reshape(CH // KBLK, KBLK)

    e0_pad = _sc_e0(table, gidx_blocks, iota_rows, euv3)

    counts_lg = _count_lg(lg_dst)
    starts_w, meta_nb, meta_st = _route_meta(counts_lg, NCHUNK, CPC)
    bsrc_b, bldst_b, bldst2_b = _fill_lg(lg_dst, lg_src, starts_w)

    outs = [e0_pad]
    for k in range(NITER + 1):
        outs.append(_msg_lg(outs[-1], e0_pad, meta_nb, meta_st,
                            bsrc_b, bldst_b))

    Wcat = jnp.concatenate(
        [W_att_root, W_att_rel, jnp.zeros((D, 14), jnp.float32)], axis=1)
    rs = [_matvec(o, Wcat) for o in outs]

    batch3d = batch.reshape(NPB, 1, PBLK)
    gouts = [
        _pool(rs[n], rs[n + 1], rs[0], batch3d, outs[n], W_gout, b_gout)
        for n in range(1, NITER + 1)
    ]
    scores = _scores(gouts, a, a_bias)
    out_fin = _fin(outs[1:NITER + 1], batch3d, scores)

    counts_e = _count_e(jnp.pad(dst, (0, EPAD - E), constant_values=2 * CH))
    starts_we, meta_nbe, meta_ste = _route_meta(counts_e, 2, 1)
    bsrc_e, bldst_e, _ = _fill_e(
        jnp.pad(dst, (0, EPAD - E), constant_values=2 * CH), dst_pad,
        starts_we)
    h_pad = jnp.pad(h, ((0, 2 * CH - N), (0, 0)))
    xo_pad = _msg_fin(out_fin, h_pad, meta_nbe, meta_ste, bsrc_e, bldst_e)
    return _final_linear(xo_pad, W_lb, b_lb)
